# baseline (speedup 1.0000x reference)
import functools

import jax
import jax.numpy as jnp
from jax.experimental import pallas as pl


_VAR0 = 0.1
_VAR1 = 0.2
_THRESH = 0.5
_NEGPOS = 3


def _row_kernel(tgt_ref, conf_ref, loc_ref, pri_ref, e_ref, v_ref, pos_ref,
                ll_ref, *, n_real, n_truth):
    i = pl.program_id(0)
    pp = pri_ref.shape[1]

    pcx = pri_ref[0:1, :]
    pcy = pri_ref[1:2, :]
    pw = pri_ref[2:3, :]
    ph = pri_ref[3:4, :]
    px1 = pcx - pw / 2.0
    py1 = pcy - ph / 2.0
    px2 = pcx + pw / 2.0
    py2 = pcy + ph / 2.0

    tx1 = tgt_ref[0, :, 0:1]
    ty1 = tgt_ref[0, :, 1:2]
    tx2 = tgt_ref[0, :, 2:3]
    ty2 = tgt_ref[0, :, 3:4]
    lab = tgt_ref[0, :, 4:5]

    iw = jnp.clip(jnp.minimum(tx2, px2) - jnp.maximum(tx1, px1), 0.0)
    ih = jnp.clip(jnp.minimum(ty2, py2) - jnp.maximum(ty1, py1), 0.0)
    inter = iw * ih
    area_a = (tx2 - tx1) * (ty2 - ty1)
    area_b = (px2 - px1) * (py2 - py1)
    ov = inter / (area_a + area_b - inter)

    tio_i = jax.lax.broadcasted_iota(jnp.int32, (n_truth, pp), 0)
    pio_i = jax.lax.broadcasted_iota(jnp.int32, (n_truth, pp), 1)
    w2 = jax.lax.bitcast_convert_type(
        jax.lax.shift_left(
            jax.lax.broadcasted_iota(jnp.int32, (1, n_truth), 1) + 127, 23),
        jnp.float32)
    hi = jax.lax.Precision.HIGHEST

    def expo(x):
        return (jax.lax.shift_right_logical(
            jax.lax.bitcast_convert_type(x, jnp.int32), 23) - 127)

    w2b = w2.astype(jnp.bfloat16)
    bto = jnp.max(ov, axis=0, keepdims=True)
    eqm = jnp.where(ov == bto, 1.0, 0.0).astype(jnp.bfloat16)
    m2 = jax.lax.dot_general(w2b, eqm, (((1,), (0,)), ((), ())),
                             preferred_element_type=jnp.float32)
    m2i = m2.astype(jnp.int32)
    bti_i = expo(jnp.bitwise_and(m2i, 0 - m2i).astype(jnp.float32))
    bpo = jnp.max(ov, axis=1, keepdims=True)
    eqp = ov == bpo
    fp_i = jnp.min(jnp.where(eqp, pio_i, pp), axis=1, keepdims=True)
    upd = jnp.where(jnp.logical_and(eqp, pio_i == fp_i), 1.0,
                    0.0).astype(jnp.bfloat16)
    movr = jax.lax.dot_general(w2b, upd, (((1,), (0,)), ((), ())),
                               preferred_element_type=jnp.float32)
    has = movr > 0.0
    bti_i = jnp.where(has, expo(movr), bti_i)
    bto = jnp.where(has, 2.0, bto)

    onehot = jnp.where(tio_i == bti_i, 1.0, 0.0)
    tgt5 = tgt_ref[0]
    mat = jax.lax.dot_general(tgt5, onehot, (((0,), (0,)), ((), ())),
                              precision=hi)
    mx1 = mat[0:1, :]
    my1 = mat[1:2, :]
    mx2 = mat[2:3, :]
    my2 = mat[3:4, :]
    labm = mat[4:5, :]

    conf_t = jnp.where(bto < _THRESH, 0.0, labm + 1.0)
    posm = conf_t > 0.0

    g_cx = ((mx1 + mx2) / 2.0 - pcx) / (_VAR0 * pw)
    g_cy = ((my1 + my2) / 2.0 - pcy) / (_VAR0 * ph)
    g_w = jnp.log((mx2 - mx1) / pw) / _VAR1
    g_h = jnp.log((my2 - my1) / ph) / _VAR1

    def sl1(d):
        ad = jnp.abs(d)
        return jnp.where(ad < 1.0, 0.5 * d * d, ad - 0.5)

    sll = (sl1(loc_ref[0, 0:1, :] - g_cx) + sl1(loc_ref[0, 1:2, :] - g_cy)
           + sl1(loc_ref[0, 2:3, :] - g_w) + sl1(loc_ref[0, 3:4, :] - g_h))
    ll_row = jnp.sum(jnp.where(posm, sll, 0.0), axis=(0, 1), keepdims=True)

    @pl.when(i == 0)
    def _init():
        ll_ref[...] = jnp.zeros((1, 1), jnp.float32)

    ll_ref[...] += ll_row

    conf = jnp.swapaxes(conf_ref[0], 0, 1)
    cmax = jnp.max(conf, axis=0, keepdims=True)
    lse = jnp.log(jnp.sum(jnp.exp(conf - cmax), axis=0, keepdims=True)) + cmax
    cio_i = jax.lax.broadcasted_iota(jnp.int32, conf.shape, 0)
    ct_i = conf_t.astype(jnp.int32)
    gath = jnp.sum(jnp.where(cio_i == ct_i, conf, 0.0), axis=0, keepdims=True)
    e = lse - gath
    v = jnp.where(posm, 0.0, e)

    e_ref[...] = e[None]
    v_ref[...] = v[None]
    pos_ref[...] = jnp.where(posm, 1.0, 0.0)[None]


def _select_kernel(e_ref, v_ref, pos_ref, ll_ref, outl_ref, outc_ref, *,
                   n_real):
    e = e_ref[...]
    v = v_ref[...]
    posf = pos_ref[...]
    b, pp = e.shape

    npos = jnp.sum(posf, axis=1, keepdims=True)
    k = jnp.minimum(float(_NEGPOS) * npos, float(n_real - 1))

    u = jax.lax.bitcast_convert_type(v, jnp.int32)

    prefix = jnp.zeros((b, 1), jnp.int32)
    rem = k
    for bit in range(30, -1, -1):
        cand_hi = jax.lax.shift_right_logical(prefix, bit) | 1
        m = jax.lax.shift_right_logical(u, bit) == cand_hi
        cnt = jnp.sum(jnp.where(m, 1.0, 0.0), axis=1, keepdims=True)
        take = cnt >= rem
        prefix = jnp.where(take, prefix | (1 << bit), prefix)
        rem = jnp.where(take, rem, rem - cnt)

    gt = u > prefix
    sum_gt = jnp.sum(jnp.where(gt, e, 0.0), axis=1, keepdims=True)

    eq = u == prefix
    li = jax.lax.broadcasted_iota(jnp.int32, (b, pp), 1).astype(jnp.float32)
    cut = jnp.zeros((b, 1), jnp.float32)
    for bit in range(14, -1, -1):
        cand = cut + float(1 << bit)
        cnt = jnp.sum(
            jnp.where(jnp.logical_and(eq, li < cand), 1.0, 0.0),
            axis=1, keepdims=True)
        cut = jnp.where(cnt <= rem, cand, cut)
    sel_eq = jnp.logical_and(jnp.logical_and(eq, li < cut), posf == 0.0)
    sum_eq = jnp.sum(jnp.where(sel_eq, e, 0.0), axis=1, keepdims=True)

    sum_pos = jnp.sum(jnp.where(posf > 0.0, e, 0.0), axis=1, keepdims=True)

    lc = jnp.sum(sum_gt + sum_eq + sum_pos, axis=0, keepdims=True)
    n_tot = jnp.sum(npos, axis=0, keepdims=True)
    outl_ref[...] = ll_ref[...] / n_tot
    outc_ref[...] = lc / n_tot


def kernel(loc_data, conf_data, priors, targets):
    b, p, c = conf_data.shape
    o = targets.shape[1]
    pp = p

    loc_t = jnp.transpose(loc_data, (0, 2, 1))
    pri_t = priors.T

    e, v, posf, ll = pl.pallas_call(
        functools.partial(_row_kernel, n_real=p, n_truth=o),
        grid=(b,),
        in_specs=[
            pl.BlockSpec((1, o, 5), lambda i: (i, 0, 0)),
            pl.BlockSpec((1, p, c), lambda i: (i, 0, 0)),
            pl.BlockSpec((1, 4, pp), lambda i: (i, 0, 0)),
            pl.BlockSpec((4, pp), lambda i: (0, 0)),
        ],
        out_specs=[
            pl.BlockSpec((1, 1, pp), lambda i: (i, 0, 0)),
            pl.BlockSpec((1, 1, pp), lambda i: (i, 0, 0)),
            pl.BlockSpec((1, 1, pp), lambda i: (i, 0, 0)),
            pl.BlockSpec((1, 1), lambda i: (0, 0)),
        ],
        out_shape=[
            jax.ShapeDtypeStruct((b, 1, pp), jnp.float32),
            jax.ShapeDtypeStruct((b, 1, pp), jnp.float32),
            jax.ShapeDtypeStruct((b, 1, pp), jnp.float32),
            jax.ShapeDtypeStruct((1, 1), jnp.float32),
        ],
    )(targets, conf_data, loc_t, pri_t)

    e = e.reshape(b, pp)
    v = v.reshape(b, pp)
    posf = posf.reshape(b, pp)

    outl, outc = pl.pallas_call(
        functools.partial(_select_kernel, n_real=p),
        grid=(1,),
        in_specs=[
            pl.BlockSpec((b, pp), lambda i: (0, 0)),
            pl.BlockSpec((b, pp), lambda i: (0, 0)),
            pl.BlockSpec((b, pp), lambda i: (0, 0)),
            pl.BlockSpec((1, 1), lambda i: (0, 0)),
        ],
        out_specs=[
            pl.BlockSpec((1, 1), lambda i: (0, 0)),
            pl.BlockSpec((1, 1), lambda i: (0, 0)),
        ],
        out_shape=[
            jax.ShapeDtypeStruct((1, 1), jnp.float32),
            jax.ShapeDtypeStruct((1, 1), jnp.float32),
        ],
    )(e, v, posf, ll)

    return outl[0, 0], outc[0, 0]


# speedup vs baseline: 1.2448x; 1.2448x over previous
import functools

import jax
import jax.numpy as jnp
from jax.experimental import pallas as pl


_VAR0 = 0.1
_VAR1 = 0.2
_THRESH = 0.5
_NEGPOS = 3


def _row_kernel(tgt_ref, conf_ref, loc_ref, pri_ref, e_ref, v_ref, pos_ref,
                ll_ref, *, n_real, n_truth):
    i = pl.program_id(0)
    pp = pri_ref.shape[1]

    pcx = pri_ref[0:1, :]
    pcy = pri_ref[1:2, :]
    pw = pri_ref[2:3, :]
    ph = pri_ref[3:4, :]
    px1 = pcx - pw / 2.0
    py1 = pcy - ph / 2.0
    px2 = pcx + pw / 2.0
    py2 = pcy + ph / 2.0

    tx1 = tgt_ref[0, :, 0:1]
    ty1 = tgt_ref[0, :, 1:2]
    tx2 = tgt_ref[0, :, 2:3]
    ty2 = tgt_ref[0, :, 3:4]
    lab = tgt_ref[0, :, 4:5]

    iw = jnp.clip(jnp.minimum(tx2, px2) - jnp.maximum(tx1, px1), 0.0)
    ih = jnp.clip(jnp.minimum(ty2, py2) - jnp.maximum(ty1, py1), 0.0)
    inter = iw * ih
    area_a = (tx2 - tx1) * (ty2 - ty1)
    area_b = (px2 - px1) * (py2 - py1)
    ov = inter / (area_a + area_b - inter)

    tio_i = jax.lax.broadcasted_iota(jnp.int32, (n_truth, pp), 0)
    pio_i = jax.lax.broadcasted_iota(jnp.int32, (n_truth, pp), 1)
    w2 = jax.lax.bitcast_convert_type(
        jax.lax.shift_left(
            jax.lax.broadcasted_iota(jnp.int32, (1, n_truth), 1) + 127, 23),
        jnp.float32)
    hi = jax.lax.Precision.HIGHEST

    def expo(x):
        return (jax.lax.shift_right_logical(
            jax.lax.bitcast_convert_type(x, jnp.int32), 23) - 127)

    w2b = w2.astype(jnp.bfloat16)
    bto = jnp.max(ov, axis=0, keepdims=True)
    eqm = jnp.where(ov == bto, 1.0, 0.0).astype(jnp.bfloat16)
    m2 = jax.lax.dot_general(w2b, eqm, (((1,), (0,)), ((), ())),
                             preferred_element_type=jnp.float32)
    m2i = m2.astype(jnp.int32)
    bti_i = expo(jnp.bitwise_and(m2i, 0 - m2i).astype(jnp.float32))
    bpo = jnp.max(ov, axis=1, keepdims=True)
    eqp = ov == bpo
    fp_i = jnp.min(jnp.where(eqp, pio_i, pp), axis=1, keepdims=True)
    upd = jnp.where(jnp.logical_and(eqp, pio_i == fp_i), 1.0,
                    0.0).astype(jnp.bfloat16)
    movr = jax.lax.dot_general(w2b, upd, (((1,), (0,)), ((), ())),
                               preferred_element_type=jnp.float32)
    has = movr > 0.0
    bti_i = jnp.where(has, expo(movr), bti_i)
    bto = jnp.where(has, 2.0, bto)

    onehot = jnp.where(tio_i == bti_i, 1.0, 0.0)
    tgt5 = tgt_ref[0]
    mat = jax.lax.dot_general(tgt5, onehot, (((0,), (0,)), ((), ())),
                              precision=hi)
    mx1 = mat[0:1, :]
    my1 = mat[1:2, :]
    mx2 = mat[2:3, :]
    my2 = mat[3:4, :]
    labm = mat[4:5, :]

    conf_t = jnp.where(bto < _THRESH, 0.0, labm + 1.0)
    posm = conf_t > 0.0

    g_cx = ((mx1 + mx2) / 2.0 - pcx) / (_VAR0 * pw)
    g_cy = ((my1 + my2) / 2.0 - pcy) / (_VAR0 * ph)
    g_w = jnp.log((mx2 - mx1) / pw) / _VAR1
    g_h = jnp.log((my2 - my1) / ph) / _VAR1

    def sl1(d):
        ad = jnp.abs(d)
        return jnp.where(ad < 1.0, 0.5 * d * d, ad - 0.5)

    sll = (sl1(loc_ref[0, 0:1, :] - g_cx) + sl1(loc_ref[0, 1:2, :] - g_cy)
           + sl1(loc_ref[0, 2:3, :] - g_w) + sl1(loc_ref[0, 3:4, :] - g_h))
    ll_row = jnp.sum(jnp.where(posm, sll, 0.0), axis=(0, 1), keepdims=True)

    @pl.when(i == 0)
    def _init():
        ll_ref[...] = jnp.zeros((1, 1), jnp.float32)

    ll_ref[...] += ll_row

    conf = conf_ref[0]
    cmax = jnp.max(conf, axis=0, keepdims=True)
    lse = jnp.log(jnp.sum(jnp.exp(conf - cmax), axis=0, keepdims=True)) + cmax
    cio_i = jax.lax.broadcasted_iota(jnp.int32, conf.shape, 0)
    ct_i = conf_t.astype(jnp.int32)
    gath = jnp.sum(jnp.where(cio_i == ct_i, conf, 0.0), axis=0, keepdims=True)
    e = lse - gath
    real = jax.lax.broadcasted_iota(jnp.int32, (1, pp), 1) < n_real
    e = jnp.where(real, e, 0.0)
    v = jnp.where(posm, 0.0, e)

    e_ref[...] = e[None]
    v_ref[...] = v[None]
    pos_ref[...] = jnp.where(posm, 1.0, 0.0)[None]


def _select_kernel(e_ref, v_ref, pos_ref, ll_ref, outl_ref, outc_ref, *,
                   n_real):
    e = e_ref[...]
    v = v_ref[...]
    posf = pos_ref[...]
    b, pp = e.shape

    npos = jnp.sum(posf, axis=1, keepdims=True)
    k = jnp.minimum(float(_NEGPOS) * npos, float(n_real - 1))

    u = jax.lax.bitcast_convert_type(v, jnp.int32)

    prefix = jnp.zeros((b, 1), jnp.int32)
    rem = k
    for bit in range(30, -1, -1):
        cand_hi = jax.lax.shift_right_logical(prefix, bit) | 1
        m = jax.lax.shift_right_logical(u, bit) == cand_hi
        cnt = jnp.sum(jnp.where(m, 1.0, 0.0), axis=1, keepdims=True)
        take = cnt >= rem
        prefix = jnp.where(take, prefix | (1 << bit), prefix)
        rem = jnp.where(take, rem, rem - cnt)

    gt = u > prefix
    sum_gt = jnp.sum(jnp.where(gt, e, 0.0), axis=1, keepdims=True)

    eq = u == prefix
    li = jax.lax.broadcasted_iota(jnp.int32, (b, pp), 1).astype(jnp.float32)
    cut = jnp.zeros((b, 1), jnp.float32)
    for bit in range(14, -1, -1):
        cand = cut + float(1 << bit)
        cnt = jnp.sum(
            jnp.where(jnp.logical_and(eq, li < cand), 1.0, 0.0),
            axis=1, keepdims=True)
        cut = jnp.where(cnt <= rem, cand, cut)
    sel_eq = jnp.logical_and(jnp.logical_and(eq, li < cut), posf == 0.0)
    sum_eq = jnp.sum(jnp.where(sel_eq, e, 0.0), axis=1, keepdims=True)

    sum_pos = jnp.sum(jnp.where(posf > 0.0, e, 0.0), axis=1, keepdims=True)

    lc = jnp.sum(sum_gt + sum_eq + sum_pos, axis=0, keepdims=True)
    n_tot = jnp.sum(npos, axis=0, keepdims=True)
    outl_ref[...] = ll_ref[...] / n_tot
    outc_ref[...] = lc / n_tot


def kernel(loc_data, conf_data, priors, targets):
    b, p, c = conf_data.shape
    o = targets.shape[1]
    pp = ((p + 127) // 128) * 128
    pad = pp - p

    conf_tr = jnp.pad(jnp.transpose(conf_data, (0, 2, 1)),
                      ((0, 0), (0, 0), (0, pad)))
    loc_t = jnp.pad(jnp.transpose(loc_data, (0, 2, 1)),
                    ((0, 0), (0, 0), (0, pad)))
    pri_t = jnp.pad(priors.T, ((0, 0), (0, pad)))

    e, v, posf, ll = pl.pallas_call(
        functools.partial(_row_kernel, n_real=p, n_truth=o),
        grid=(b,),
        in_specs=[
            pl.BlockSpec((1, o, 5), lambda i: (i, 0, 0)),
            pl.BlockSpec((1, c, pp), lambda i: (i, 0, 0)),
            pl.BlockSpec((1, 4, pp), lambda i: (i, 0, 0)),
            pl.BlockSpec((4, pp), lambda i: (0, 0)),
        ],
        out_specs=[
            pl.BlockSpec((1, 1, pp), lambda i: (i, 0, 0)),
            pl.BlockSpec((1, 1, pp), lambda i: (i, 0, 0)),
            pl.BlockSpec((1, 1, pp), lambda i: (i, 0, 0)),
            pl.BlockSpec((1, 1), lambda i: (0, 0)),
        ],
        out_shape=[
            jax.ShapeDtypeStruct((b, 1, pp), jnp.float32),
            jax.ShapeDtypeStruct((b, 1, pp), jnp.float32),
            jax.ShapeDtypeStruct((b, 1, pp), jnp.float32),
            jax.ShapeDtypeStruct((1, 1), jnp.float32),
        ],
    )(targets, conf_tr, loc_t, pri_t)

    e = e.reshape(b, pp)
    v = v.reshape(b, pp)
    posf = posf.reshape(b, pp)

    outl, outc = pl.pallas_call(
        functools.partial(_select_kernel, n_real=p),
        grid=(1,),
        in_specs=[
            pl.BlockSpec((b, pp), lambda i: (0, 0)),
            pl.BlockSpec((b, pp), lambda i: (0, 0)),
            pl.BlockSpec((b, pp), lambda i: (0, 0)),
            pl.BlockSpec((1, 1), lambda i: (0, 0)),
        ],
        out_specs=[
            pl.BlockSpec((1, 1), lambda i: (0, 0)),
            pl.BlockSpec((1, 1), lambda i: (0, 0)),
        ],
        out_shape=[
            jax.ShapeDtypeStruct((1, 1), jnp.float32),
            jax.ShapeDtypeStruct((1, 1), jnp.float32),
        ],
    )(e, v, posf, ll)

    return outl[0, 0], outc[0, 0]


# speedup vs baseline: 1.2751x; 1.0243x over previous
import functools

import jax
import jax.numpy as jnp
from jax.experimental import pallas as pl
from jax.experimental.pallas import tpu as pltpu


_VAR0 = 0.1
_VAR1 = 0.2
_THRESH = 0.5
_NEGPOS = 3


def _row_kernel(tgt_ref, conf_ref, loc_ref, pri_ref, e_ref, v_ref, pos_ref,
                ll_ref, *, n_real, n_truth):
    i = pl.program_id(0)
    pp = pri_ref.shape[1]

    pcx = pri_ref[0:1, :]
    pcy = pri_ref[1:2, :]
    pw = pri_ref[2:3, :]
    ph = pri_ref[3:4, :]
    px1 = pcx - pw / 2.0
    py1 = pcy - ph / 2.0
    px2 = pcx + pw / 2.0
    py2 = pcy + ph / 2.0

    tx1 = tgt_ref[0, :, 0:1]
    ty1 = tgt_ref[0, :, 1:2]
    tx2 = tgt_ref[0, :, 2:3]
    ty2 = tgt_ref[0, :, 3:4]
    lab = tgt_ref[0, :, 4:5]

    iw = jnp.clip(jnp.minimum(tx2, px2) - jnp.maximum(tx1, px1), 0.0)
    ih = jnp.clip(jnp.minimum(ty2, py2) - jnp.maximum(ty1, py1), 0.0)
    inter = iw * ih
    area_a = (tx2 - tx1) * (ty2 - ty1)
    area_b = (px2 - px1) * (py2 - py1)
    ov = inter / (area_a + area_b - inter)

    tio_i = jax.lax.broadcasted_iota(jnp.int32, (n_truth, pp), 0)
    pio_i = jax.lax.broadcasted_iota(jnp.int32, (n_truth, pp), 1)
    w2 = jax.lax.bitcast_convert_type(
        jax.lax.shift_left(
            jax.lax.broadcasted_iota(jnp.int32, (1, n_truth), 1) + 127, 23),
        jnp.float32)
    hi = jax.lax.Precision.HIGHEST

    def expo(x):
        return (jax.lax.shift_right_logical(
            jax.lax.bitcast_convert_type(x, jnp.int32), 23) - 127)

    w2b = w2.astype(jnp.bfloat16)
    bto = jnp.max(ov, axis=0, keepdims=True)
    eqm = jnp.where(ov == bto, 1.0, 0.0).astype(jnp.bfloat16)
    m2 = jax.lax.dot_general(w2b, eqm, (((1,), (0,)), ((), ())),
                             preferred_element_type=jnp.float32)
    m2i = m2.astype(jnp.int32)
    bti_i = expo(jnp.bitwise_and(m2i, 0 - m2i).astype(jnp.float32))
    bpo = jnp.max(ov, axis=1, keepdims=True)
    eqp = ov == bpo
    fp_i = jnp.min(jnp.where(eqp, pio_i, pp), axis=1, keepdims=True)
    upd = jnp.where(jnp.logical_and(eqp, pio_i == fp_i), 1.0,
                    0.0).astype(jnp.bfloat16)
    movr = jax.lax.dot_general(w2b, upd, (((1,), (0,)), ((), ())),
                               preferred_element_type=jnp.float32)
    has = movr > 0.0
    bti_i = jnp.where(has, expo(movr), bti_i)
    bto = jnp.where(has, 2.0, bto)

    onehot = jnp.where(tio_i == bti_i, 1.0, 0.0)
    tgt5 = tgt_ref[0]
    mat = jax.lax.dot_general(tgt5, onehot, (((0,), (0,)), ((), ())),
                              precision=hi)
    mx1 = mat[0:1, :]
    my1 = mat[1:2, :]
    mx2 = mat[2:3, :]
    my2 = mat[3:4, :]
    labm = mat[4:5, :]

    conf_t = jnp.where(bto < _THRESH, 0.0, labm + 1.0)
    posm = conf_t > 0.0

    g_cx = ((mx1 + mx2) / 2.0 - pcx) / (_VAR0 * pw)
    g_cy = ((my1 + my2) / 2.0 - pcy) / (_VAR0 * ph)
    g_w = jnp.log((mx2 - mx1) / pw) / _VAR1
    g_h = jnp.log((my2 - my1) / ph) / _VAR1

    def sl1(d):
        ad = jnp.abs(d)
        return jnp.where(ad < 1.0, 0.5 * d * d, ad - 0.5)

    sll = (sl1(loc_ref[0, 0:1, :] - g_cx) + sl1(loc_ref[0, 1:2, :] - g_cy)
           + sl1(loc_ref[0, 2:3, :] - g_w) + sl1(loc_ref[0, 3:4, :] - g_h))
    ll_row = jnp.sum(jnp.where(posm, sll, 0.0), axis=(0, 1), keepdims=True)
    ll_ref[...] = ll_row[None]

    conf = conf_ref[0]
    cmax = jnp.max(conf, axis=0, keepdims=True)
    lse = jnp.log(jnp.sum(jnp.exp(conf - cmax), axis=0, keepdims=True)) + cmax
    cio_i = jax.lax.broadcasted_iota(jnp.int32, conf.shape, 0)
    ct_i = conf_t.astype(jnp.int32)
    gath = jnp.sum(jnp.where(cio_i == ct_i, conf, 0.0), axis=0, keepdims=True)
    e = lse - gath
    real = jax.lax.broadcasted_iota(jnp.int32, (1, pp), 1) < n_real
    e = jnp.where(real, e, 0.0)
    v = jnp.where(posm, 0.0, e)

    e_ref[...] = e[None]
    v_ref[...] = v[None]
    pos_ref[...] = jnp.where(posm, 1.0, 0.0)[None]


def _select_kernel(e_ref, v_ref, pos_ref, ll_ref, outl_ref, outc_ref, *,
                   n_real):
    e = e_ref[...]
    v = v_ref[...]
    posf = pos_ref[...]
    b, pp = e.shape
    lltot = jnp.sum(ll_ref[...], axis=0, keepdims=True)

    npos = jnp.sum(posf, axis=1, keepdims=True)
    k = jnp.minimum(float(_NEGPOS) * npos, float(n_real - 1))

    u = jax.lax.bitcast_convert_type(v, jnp.int32)

    prefix = jnp.zeros((b, 1), jnp.int32)
    rem = k
    for bit in range(30, -1, -1):
        cand_hi = jax.lax.shift_right_logical(prefix, bit) | 1
        m = jax.lax.shift_right_logical(u, bit) == cand_hi
        cnt = jnp.sum(jnp.where(m, 1.0, 0.0), axis=1, keepdims=True)
        take = cnt >= rem
        prefix = jnp.where(take, prefix | (1 << bit), prefix)
        rem = jnp.where(take, rem, rem - cnt)

    gt = u > prefix
    sum_gt = jnp.sum(jnp.where(gt, e, 0.0), axis=1, keepdims=True)

    eq = u == prefix
    li = jax.lax.broadcasted_iota(jnp.int32, (b, pp), 1).astype(jnp.float32)
    cut = jnp.zeros((b, 1), jnp.float32)
    for bit in range(14, -1, -1):
        cand = cut + float(1 << bit)
        cnt = jnp.sum(
            jnp.where(jnp.logical_and(eq, li < cand), 1.0, 0.0),
            axis=1, keepdims=True)
        cut = jnp.where(cnt <= rem, cand, cut)
    sel_eq = jnp.logical_and(jnp.logical_and(eq, li < cut), posf == 0.0)
    sum_eq = jnp.sum(jnp.where(sel_eq, e, 0.0), axis=1, keepdims=True)

    sum_pos = jnp.sum(jnp.where(posf > 0.0, e, 0.0), axis=1, keepdims=True)

    lc = jnp.sum(sum_gt + sum_eq + sum_pos, axis=0, keepdims=True)
    n_tot = jnp.sum(npos, axis=0, keepdims=True)
    outl_ref[...] = lltot / n_tot
    outc_ref[...] = lc / n_tot


def kernel(loc_data, conf_data, priors, targets):
    b, p, c = conf_data.shape
    o = targets.shape[1]
    pp = ((p + 127) // 128) * 128
    pad = pp - p

    conf_tr = jnp.pad(jnp.transpose(conf_data, (0, 2, 1)),
                      ((0, 0), (0, 0), (0, pad)))
    loc_t = jnp.pad(jnp.transpose(loc_data, (0, 2, 1)),
                    ((0, 0), (0, 0), (0, pad)))
    pri_t = jnp.pad(priors.T, ((0, 0), (0, pad)))

    e, v, posf, ll = pl.pallas_call(
        functools.partial(_row_kernel, n_real=p, n_truth=o),
        grid=(b,),
        in_specs=[
            pl.BlockSpec((1, o, 5), lambda i: (i, 0, 0)),
            pl.BlockSpec((1, c, pp), lambda i: (i, 0, 0)),
            pl.BlockSpec((1, 4, pp), lambda i: (i, 0, 0)),
            pl.BlockSpec((4, pp), lambda i: (0, 0)),
        ],
        out_specs=[
            pl.BlockSpec((1, 1, pp), lambda i: (i, 0, 0)),
            pl.BlockSpec((1, 1, pp), lambda i: (i, 0, 0)),
            pl.BlockSpec((1, 1, pp), lambda i: (i, 0, 0)),
            pl.BlockSpec((1, 1, 1), lambda i: (i, 0, 0)),
        ],
        out_shape=[
            jax.ShapeDtypeStruct((b, 1, pp), jnp.float32),
            jax.ShapeDtypeStruct((b, 1, pp), jnp.float32),
            jax.ShapeDtypeStruct((b, 1, pp), jnp.float32),
            jax.ShapeDtypeStruct((b, 1, 1), jnp.float32),
        ],
        compiler_params=pltpu.CompilerParams(
            dimension_semantics=("parallel",)),
    )(targets, conf_tr, loc_t, pri_t)

    e = e.reshape(b, pp)
    v = v.reshape(b, pp)
    posf = posf.reshape(b, pp)
    ll = ll.reshape(b, 1)

    outl, outc = pl.pallas_call(
        functools.partial(_select_kernel, n_real=p),
        grid=(1,),
        in_specs=[
            pl.BlockSpec((b, pp), lambda i: (0, 0)),
            pl.BlockSpec((b, pp), lambda i: (0, 0)),
            pl.BlockSpec((b, pp), lambda i: (0, 0)),
            pl.BlockSpec((b, 1), lambda i: (0, 0)),
        ],
        out_specs=[
            pl.BlockSpec((1, 1), lambda i: (0, 0)),
            pl.BlockSpec((1, 1), lambda i: (0, 0)),
        ],
        out_shape=[
            jax.ShapeDtypeStruct((1, 1), jnp.float32),
            jax.ShapeDtypeStruct((1, 1), jnp.float32),
        ],
    )(e, v, posf, ll)

    return outl[0, 0], outc[0, 0]


# speedup vs baseline: 1.4377x; 1.1276x over previous
import functools

import jax
import jax.numpy as jnp
from jax.experimental import pallas as pl
from jax.experimental.pallas import tpu as pltpu


_VAR0 = 0.1
_VAR1 = 0.2
_THRESH = 0.5
_NEGPOS = 3


def _row_kernel(tgt_ref, conf_ref, loc_ref, pri_ref, e_ref, v_ref, pos_ref,
                ll_ref, *, n_real, n_truth):
    i = pl.program_id(0)
    pp = pri_ref.shape[1]

    pcx = pri_ref[0:1, :]
    pcy = pri_ref[1:2, :]
    pw = pri_ref[2:3, :]
    ph = pri_ref[3:4, :]
    px1 = pcx - pw / 2.0
    py1 = pcy - ph / 2.0
    px2 = pcx + pw / 2.0
    py2 = pcy + ph / 2.0

    tx1 = tgt_ref[0, :, 0:1]
    ty1 = tgt_ref[0, :, 1:2]
    tx2 = tgt_ref[0, :, 2:3]
    ty2 = tgt_ref[0, :, 3:4]
    lab = tgt_ref[0, :, 4:5]

    iw = jnp.clip(jnp.minimum(tx2, px2) - jnp.maximum(tx1, px1), 0.0)
    ih = jnp.clip(jnp.minimum(ty2, py2) - jnp.maximum(ty1, py1), 0.0)
    inter = iw * ih
    area_a = (tx2 - tx1) * (ty2 - ty1)
    area_b = (px2 - px1) * (py2 - py1)
    ov = inter / (area_a + area_b - inter)

    tio_i = jax.lax.broadcasted_iota(jnp.int32, (n_truth, pp), 0)
    pio_i = jax.lax.broadcasted_iota(jnp.int32, (n_truth, pp), 1)
    w2 = jax.lax.bitcast_convert_type(
        jax.lax.shift_left(
            jax.lax.broadcasted_iota(jnp.int32, (1, n_truth), 1) + 127, 23),
        jnp.float32)
    hi = jax.lax.Precision.HIGHEST

    def expo(x):
        return (jax.lax.shift_right_logical(
            jax.lax.bitcast_convert_type(x, jnp.int32), 23) - 127)

    w2b = w2.astype(jnp.bfloat16)
    bto = jnp.max(ov, axis=0, keepdims=True)
    eqm = jnp.where(ov == bto, 1.0, 0.0).astype(jnp.bfloat16)
    m2 = jax.lax.dot_general(w2b, eqm, (((1,), (0,)), ((), ())),
                             preferred_element_type=jnp.float32)
    m2i = m2.astype(jnp.int32)
    bti_i = expo(jnp.bitwise_and(m2i, 0 - m2i).astype(jnp.float32))
    bpo = jnp.max(ov, axis=1, keepdims=True)
    eqp = ov == bpo
    fp_i = jnp.min(jnp.where(eqp, pio_i, pp), axis=1, keepdims=True)
    upd = jnp.where(jnp.logical_and(eqp, pio_i == fp_i), 1.0,
                    0.0).astype(jnp.bfloat16)
    movr = jax.lax.dot_general(w2b, upd, (((1,), (0,)), ((), ())),
                               preferred_element_type=jnp.float32)
    has = movr > 0.0
    bti_i = jnp.where(has, expo(movr), bti_i)
    bto = jnp.where(has, 2.0, bto)

    onehot = jnp.where(tio_i == bti_i, 1.0, 0.0)
    tgt5 = tgt_ref[0]
    mat = jax.lax.dot_general(tgt5, onehot, (((0,), (0,)), ((), ())),
                              precision=hi)
    mx1 = mat[0:1, :]
    my1 = mat[1:2, :]
    mx2 = mat[2:3, :]
    my2 = mat[3:4, :]
    labm = mat[4:5, :]

    conf_t = jnp.where(bto < _THRESH, 0.0, labm + 1.0)
    posm = conf_t > 0.0

    g_cx = ((mx1 + mx2) / 2.0 - pcx) / (_VAR0 * pw)
    g_cy = ((my1 + my2) / 2.0 - pcy) / (_VAR0 * ph)
    g_w = jnp.log((mx2 - mx1) / pw) / _VAR1
    g_h = jnp.log((my2 - my1) / ph) / _VAR1

    def sl1(d):
        ad = jnp.abs(d)
        return jnp.where(ad < 1.0, 0.5 * d * d, ad - 0.5)

    sll = (sl1(loc_ref[0, 0:1, :] - g_cx) + sl1(loc_ref[0, 1:2, :] - g_cy)
           + sl1(loc_ref[0, 2:3, :] - g_w) + sl1(loc_ref[0, 3:4, :] - g_h))
    ll_row = jnp.sum(jnp.where(posm, sll, 0.0), axis=(0, 1), keepdims=True)
    ll_ref[...] = ll_row[None]

    conf = conf_ref[0]
    cmax = jnp.max(conf, axis=0, keepdims=True)
    lse = jnp.log(jnp.sum(jnp.exp(conf - cmax), axis=0, keepdims=True)) + cmax
    cio_i = jax.lax.broadcasted_iota(jnp.int32, conf.shape, 0)
    ct_i = conf_t.astype(jnp.int32)
    gath = jnp.sum(jnp.where(cio_i == ct_i, conf, 0.0), axis=0, keepdims=True)
    e = lse - gath
    real = jax.lax.broadcasted_iota(jnp.int32, (1, pp), 1) < n_real
    e = jnp.where(real, e, 0.0)
    v = jnp.where(posm, 0.0, e)

    e_ref[...] = e[None]
    v_ref[...] = v[None]
    pos_ref[...] = jnp.where(posm, 1.0, 0.0)[None]


def _select_kernel(e_ref, v_ref, pos_ref, ll_ref, outl_ref, outc_ref, *,
                   n_real):
    e = e_ref[...]
    v = v_ref[...]
    posf = pos_ref[...]
    b, pp = e.shape
    lltot = jnp.sum(ll_ref[...], axis=0, keepdims=True)

    npos = jnp.sum(posf, axis=1, keepdims=True)
    k = jnp.minimum(float(_NEGPOS) * npos, float(n_real - 1))

    u = jax.lax.bitcast_convert_type(v, jnp.int32)

    prefix = jnp.zeros((b, 1), jnp.int32)
    rem = k
    for bit in range(30, -1, -1):
        cand_hi = jax.lax.shift_right_logical(prefix, bit) | 1
        m = jax.lax.shift_right_logical(u, bit) == cand_hi
        cnt = jnp.sum(jnp.where(m, 1.0, 0.0), axis=1, keepdims=True)
        take = cnt >= rem
        prefix = jnp.where(take, prefix | (1 << bit), prefix)
        rem = jnp.where(take, rem, rem - cnt)

    gt = u > prefix
    sum_gt = jnp.sum(jnp.where(gt, e, 0.0), axis=1, keepdims=True)

    eq = u == prefix
    li = jax.lax.broadcasted_iota(jnp.int32, (b, pp), 1).astype(jnp.float32)
    cut = jnp.zeros((b, 1), jnp.float32)
    for bit in range(14, -1, -1):
        cand = cut + float(1 << bit)
        cnt = jnp.sum(
            jnp.where(jnp.logical_and(eq, li < cand), 1.0, 0.0),
            axis=1, keepdims=True)
        cut = jnp.where(cnt <= rem, cand, cut)
    sel_eq = jnp.logical_and(jnp.logical_and(eq, li < cut), posf == 0.0)
    sum_eq = jnp.sum(jnp.where(sel_eq, e, 0.0), axis=1, keepdims=True)

    sum_pos = jnp.sum(jnp.where(posf > 0.0, e, 0.0), axis=1, keepdims=True)

    lc = jnp.sum(sum_gt + sum_eq + sum_pos, axis=0, keepdims=True)
    n_tot = jnp.sum(npos, axis=0, keepdims=True)
    outl_ref[...] = lltot / n_tot
    outc_ref[...] = lc / n_tot


def kernel(loc_data, conf_data, priors, targets):
    b, p, c = conf_data.shape
    o = targets.shape[1]
    pp = p

    conf_tr = jnp.transpose(conf_data, (0, 2, 1))
    loc_t = jnp.transpose(loc_data, (0, 2, 1))
    pri_t = priors.T

    e, v, posf, ll = pl.pallas_call(
        functools.partial(_row_kernel, n_real=p, n_truth=o),
        grid=(b,),
        in_specs=[
            pl.BlockSpec((1, o, 5), lambda i: (i, 0, 0)),
            pl.BlockSpec((1, c, pp), lambda i: (i, 0, 0)),
            pl.BlockSpec((1, 4, pp), lambda i: (i, 0, 0)),
            pl.BlockSpec((4, pp), lambda i: (0, 0)),
        ],
        out_specs=[
            pl.BlockSpec((1, 1, pp), lambda i: (i, 0, 0)),
            pl.BlockSpec((1, 1, pp), lambda i: (i, 0, 0)),
            pl.BlockSpec((1, 1, pp), lambda i: (i, 0, 0)),
            pl.BlockSpec((1, 1, 1), lambda i: (i, 0, 0)),
        ],
        out_shape=[
            jax.ShapeDtypeStruct((b, 1, pp), jnp.float32),
            jax.ShapeDtypeStruct((b, 1, pp), jnp.float32),
            jax.ShapeDtypeStruct((b, 1, pp), jnp.float32),
            jax.ShapeDtypeStruct((b, 1, 1), jnp.float32),
        ],
        compiler_params=pltpu.CompilerParams(
            dimension_semantics=("parallel",)),
    )(targets, conf_tr, loc_t, pri_t)

    e = e.reshape(b, pp)
    v = v.reshape(b, pp)
    posf = posf.reshape(b, pp)
    ll = ll.reshape(b, 1)

    outl, outc = pl.pallas_call(
        functools.partial(_select_kernel, n_real=p),
        grid=(1,),
        in_specs=[
            pl.BlockSpec((b, pp), lambda i: (0, 0)),
            pl.BlockSpec((b, pp), lambda i: (0, 0)),
            pl.BlockSpec((b, pp), lambda i: (0, 0)),
            pl.BlockSpec((b, 1), lambda i: (0, 0)),
        ],
        out_specs=[
            pl.BlockSpec((1, 1), lambda i: (0, 0)),
            pl.BlockSpec((1, 1), lambda i: (0, 0)),
        ],
        out_shape=[
            jax.ShapeDtypeStruct((1, 1), jnp.float32),
            jax.ShapeDtypeStruct((1, 1), jnp.float32),
        ],
    )(e, v, posf, ll)

    return outl[0, 0], outc[0, 0]
